# baseline (speedup 1.0000x reference)
import jax
import jax.numpy as jnp
from jax import lax
from jax.experimental import pallas as pl
from jax.experimental.pallas import tpu as pltpu
from jax.experimental.pallas import tpu_sc as plsc

N = 100000
K = 128
NC = 2
NS = 16
NW = NC * NS
L = 16
WPR = 3136
CH = 112
NCHUNK = WPR // CH
NG = CH // L


def _compute_chunk(bufs, tmp, outv, off):
    bs, bp, bo = bufs
    iota = lax.iota(jnp.int32, L)
    idx_base = iota * L

    def group_body(g, carry):
        base = g * L
        for r16 in range(L):
            r = base + r16
            acc = None
            for j in range(8):
                s = bs[r, 0, pl.ds(j * L, L)]
                p = bp[r, 0, pl.ds(j * L, L)]
                o = bo[r, 0, pl.ds(j * L, L)]
                prod = s * p * o
                acc = prod if acc is None else acc + prod
            tmp[pl.ds(r16 * L, L)] = acc
        colsum = None
        for c in range(L):
            v = plsc.load_gather(tmp, [idx_base + c])
            colsum = v if colsum is None else colsum + v
        outv[pl.ds(off + base, L)] = colsum
        return carry

    lax.fori_loop(0, NG, group_body, 0)


def _body(x_hbm, out_hbm, s0, p0, o0, s1, p1, o1, tmp, outv, sem0, sem1):
    wid = lax.axis_index("s") * NC + lax.axis_index("c")
    start = jnp.minimum(wid * WPR, N - WPR)
    rings = ((s0, p0, o0), (s1, p1, o1))
    sems = (sem0, sem1)

    def start_chunk(ci, slot):
        rows = pl.ds(start + ci * CH, CH)
        for c in range(3):
            pltpu.async_copy(
                x_hbm.at[rows, pl.ds(c, 1)], rings[slot][c], sems[slot]
            )

    def wait_chunk(slot):
        for c in range(3):
            pltpu.make_async_copy(
                x_hbm.at[pl.ds(0, CH), pl.ds(c, 1)], rings[slot][c], sems[slot]
            ).wait()

    start_chunk(0, 0)

    def pair_body(i, carry):
        ci = 2 * i
        start_chunk(ci + 1, 1)
        wait_chunk(0)
        _compute_chunk(rings[0], tmp, outv, ci * CH)
        start_chunk(ci + 2, 0)
        wait_chunk(1)
        _compute_chunk(rings[1], tmp, outv, (ci + 1) * CH)
        return carry

    lax.fori_loop(0, NCHUNK // 2 - 1, pair_body, 0)

    ci = NCHUNK - 2
    start_chunk(ci + 1, 1)
    wait_chunk(0)
    _compute_chunk(rings[0], tmp, outv, ci * CH)
    wait_chunk(1)
    _compute_chunk(rings[1], tmp, outv, (ci + 1) * CH)

    pltpu.sync_copy(outv, out_hbm.at[pl.ds(start, WPR)])


@jax.jit
def kernel(triples):
    plane = pltpu.VMEM((CH, 1, K), jnp.float32)
    k = pl.kernel(
        _body,
        out_type=jax.ShapeDtypeStruct((N,), jnp.float32),
        mesh=plsc.VectorSubcoreMesh(core_axis_name="c", subcore_axis_name="s"),
        scratch_types=[
            plane, plane, plane, plane, plane, plane,
            pltpu.VMEM((L * L,), jnp.float32),
            pltpu.VMEM((WPR,), jnp.float32),
            pltpu.SemaphoreType.DMA,
            pltpu.SemaphoreType.DMA,
        ],
        compiler_params=pltpu.CompilerParams(needs_layout_passes=False),
    )
    return k(triples)
